# baseline (speedup 1.0000x reference)
import functools

import jax
import jax.numpy as jnp
from jax import lax
from jax.experimental import pallas as pl
from jax.experimental.pallas import tpu as pltpu
from jax.experimental.pallas import tpu_sc as plsc

N = 10000
E = 320000
D = 128

NC = 2
NS = 16
NW = NC * NS

EPT = E // NW
C = 80
NCHUNK = EPT // C

RCHUNK = C
NROWCH = N // RCHUNK

NBUF = 3
NOUT = NCHUNK // NBUF


def _mask_body(h_ref, f_ref, e_ref, hm_ref, src_ref, dst_ref):
    hm_ref[...] = h_ref[...] * f_ref[...]
    src_ref[...] = e_ref[0, :]
    dst_ref[...] = e_ref[1, :]


def _masked_rows(H, frag_col, edge_index_i32):
    grid = 5
    blk = N // grid
    eblk = 65536
    return pl.pallas_call(
        _mask_body,
        grid=(grid,),
        in_specs=[
            pl.BlockSpec((blk, D), lambda i: (i, 0)),
            pl.BlockSpec((blk, 1), lambda i: (i, 0)),
            pl.BlockSpec((2, eblk), lambda i: (0, i)),
        ],
        out_specs=[
            pl.BlockSpec((blk, D), lambda i: (i, 0)),
            pl.BlockSpec((eblk,), lambda i: (i,)),
            pl.BlockSpec((eblk,), lambda i: (i,)),
        ],
        out_shape=[
            jax.ShapeDtypeStruct((N, D), jnp.float32),
            jax.ShapeDtypeStruct((E,), jnp.int32),
            jax.ShapeDtypeStruct((E,), jnp.int32),
        ],
    )(H, frag_col, edge_index_i32)


def _sc_body(hm_hbm, src_hbm, dst_hbm, out_hbm,
             acc, r0, r1, r2, d0, d1, d2, sidx_all,
             isem, gsem, ssem):
    rows = [r0, r1, r2]
    didx = [d0, d1, d2]
    rbuf = rows[0]
    cid = lax.axis_index("c")
    sid = lax.axis_index("s")
    wid = cid * NS + sid
    base = wid * EPT

    def _zero_vec(i, _):
        r = i // (D // 16)
        c = i % (D // 16)
        rbuf[r, pl.ds(c * 16, 16)] = jnp.zeros((16,), jnp.float32)
        return _
    lax.fori_loop(0, RCHUNK * (D // 16), _zero_vec, None)

    def _zero_fire(j, _):
        k = sid + j * NS
        @pl.when(k < NROWCH)
        def _():
            pltpu.async_copy(rbuf, acc.at[pl.ds(k * RCHUNK, RCHUNK), :],
                             gsem.at[0])
        return _
    lax.fori_loop(0, (NROWCH + NS - 1) // NS, _zero_fire, None)

    def _zero_drain(j, _):
        k = sid + j * NS
        @pl.when(k < NROWCH)
        def _():
            pltpu.make_async_copy(
                rbuf, acc.at[pl.ds(k * RCHUNK, RCHUNK), :],
                gsem.at[0]).wait()
        return _
    lax.fori_loop(0, (NROWCH + NS - 1) // NS, _zero_drain, None)

    pltpu.sync_copy(src_hbm.at[pl.ds(base, EPT)], sidx_all)
    plsc.subcore_barrier()

    def _fire_didx(g, b):
        pltpu.async_copy(dst_hbm.at[pl.ds(base + g * C, C)], didx[b],
                         isem.at[b])

    def _wait_didx(b):
        pltpu.make_async_copy(
            dst_hbm.at[pl.ds(0, C)], didx[b], isem.at[b]).wait()

    def _fire_gather(g, b):
        pltpu.async_copy(
            hm_hbm.at[sidx_all.at[pl.ds(g * C, C)]], rows[b], gsem.at[b])

    def _wait_gather(g, b):
        pltpu.make_async_copy(
            hm_hbm.at[sidx_all.at[pl.ds(g * C, C)]], rows[b],
            gsem.at[b]).wait()

    def _fire_scatter(b):
        pltpu.async_copy(rows[b], acc.at[didx[b]], ssem.at[b], add=True)

    def _wait_scatter(b):
        pltpu.make_async_copy(
            rows[b], acc.at[didx[b]], ssem.at[b]).wait()

    for b in range(NBUF):
        _fire_didx(b, b)
        _fire_gather(b, b)

    def _outer(t, _):
        for b in range(NBUF):
            _wait_gather(t * NBUF + b, b)
            _wait_didx(b)
            _fire_scatter(b)
        for b in range(NBUF):
            g2 = (t + 1) * NBUF + b
            @pl.when(g2 < NCHUNK)
            def _():
                _wait_scatter(b)
                _fire_didx(g2, b)
                _fire_gather(g2, b)
        return _
    lax.fori_loop(0, NOUT, _outer, None)

    for b in range(NCHUNK - NOUT * NBUF):
        _wait_gather(NOUT * NBUF + b, b)
        _wait_didx(b)
        _fire_scatter(b)
    for b in range(NBUF):
        _wait_scatter(b)
    plsc.subcore_barrier()

    def _dump_fire(j, _):
        k = sid + j * NS
        @pl.when(k < NROWCH)
        def _():
            rr = k * RCHUNK
            pltpu.async_copy(acc.at[pl.ds(rr, RCHUNK), :],
                             out_hbm.at[cid, pl.ds(rr, RCHUNK), :],
                             gsem.at[0])
        return _
    lax.fori_loop(0, (NROWCH + NS - 1) // NS, _dump_fire, None)

    def _dump_drain(j, _):
        k = sid + j * NS
        @pl.when(k < NROWCH)
        def _():
            rr = k * RCHUNK
            pltpu.make_async_copy(
                acc.at[pl.ds(rr, RCHUNK), :],
                out_hbm.at[cid, pl.ds(rr, RCHUNK), :], gsem.at[0]).wait()
        return _
    lax.fori_loop(0, (NROWCH + NS - 1) // NS, _dump_drain, None)


def _sc_scatter(Hm, src_i32, dst_i32):
    mesh = plsc.VectorSubcoreMesh(core_axis_name="c", subcore_axis_name="s")
    f = functools.partial(
        pl.kernel,
        out_type=jax.ShapeDtypeStruct((NC, N, D), jnp.float32),
        mesh=mesh,
        scratch_types=[
            pltpu.VMEM_SHARED((N, D), jnp.float32),
        ] + [pltpu.VMEM((C, D), jnp.float32)] * NBUF + [
        ] + [pltpu.VMEM((C,), jnp.int32)] * NBUF + [
            pltpu.VMEM((EPT,), jnp.int32),
            pltpu.SemaphoreType.DMA((NBUF,)),
            pltpu.SemaphoreType.DMA((NBUF,)),
            pltpu.SemaphoreType.DMA((NBUF,)),
        ],
    )(_sc_body)
    return f(Hm, src_i32, dst_i32)


def _finish_body(h_ref, f_ref, p0_ref, p1_ref, w_ref, out_ref):
    agg = p0_ref[0] + p1_ref[0]
    y = lax.dot_general(agg, w_ref[...], (((1,), (1,)), ((), ())),
                        preferred_element_type=jnp.float32)
    out_ref[...] = h_ref[...] + f_ref[...] * y


def _finish(H, frag_col, P, W):
    grid = 5
    blk = N // grid
    return pl.pallas_call(
        _finish_body,
        grid=(grid,),
        in_specs=[
            pl.BlockSpec((blk, D), lambda i: (i, 0)),
            pl.BlockSpec((blk, 1), lambda i: (i, 0)),
            pl.BlockSpec((1, blk, D), lambda i: (0, i, 0)),
            pl.BlockSpec((1, blk, D), lambda i: (1, i, 0)),
            pl.BlockSpec((D, D), lambda i: (0, 0)),
        ],
        out_specs=pl.BlockSpec((blk, D), lambda i: (i, 0)),
        out_shape=jax.ShapeDtypeStruct((N, D), jnp.float32),
    )(H, frag_col, P, P, W)


def kernel(H, edge_index, frag_mask, W):
    frag_col = frag_mask.reshape(N, 1).astype(jnp.float32)
    ei = edge_index.astype(jnp.int32)
    Hm, src, dst = _masked_rows(H, frag_col, ei)
    P = _sc_scatter(Hm, src, dst)
    return _finish(H, frag_col, P, W)


# speedup vs baseline: 1.0829x; 1.0829x over previous
import functools

import jax
import jax.numpy as jnp
from jax import lax
from jax.experimental import pallas as pl
from jax.experimental.pallas import tpu as pltpu
from jax.experimental.pallas import tpu_sc as plsc

N = 10000
E = 320000
D = 128

NC = 2
NS = 16
NW = NC * NS

EPT = E // NW
C = 40
NCHUNK = EPT // C

RCHUNK = C
NROWCH = N // RCHUNK

NBUF = 7
NOUT = NCHUNK // NBUF


def _mask_body(h_ref, f_ref, e_ref, hm_ref, src_ref, dst_ref):
    hm_ref[...] = h_ref[...] * f_ref[...]
    src_ref[...] = e_ref[0, :]
    dst_ref[...] = e_ref[1, :]


def _masked_rows(H, frag_col, edge_index_i32):
    grid = 5
    blk = N // grid
    eblk = 65536
    return pl.pallas_call(
        _mask_body,
        grid=(grid,),
        in_specs=[
            pl.BlockSpec((blk, D), lambda i: (i, 0)),
            pl.BlockSpec((blk, 1), lambda i: (i, 0)),
            pl.BlockSpec((2, eblk), lambda i: (0, i)),
        ],
        out_specs=[
            pl.BlockSpec((blk, D), lambda i: (i, 0)),
            pl.BlockSpec((eblk,), lambda i: (i,)),
            pl.BlockSpec((eblk,), lambda i: (i,)),
        ],
        out_shape=[
            jax.ShapeDtypeStruct((N, D), jnp.float32),
            jax.ShapeDtypeStruct((E,), jnp.int32),
            jax.ShapeDtypeStruct((E,), jnp.int32),
        ],
    )(H, frag_col, edge_index_i32)


def _sc_body(hm_hbm, src_hbm, dst_hbm, out_hbm,
             acc, r0, r1, r2, r3, r4, r5, r6, d0, d1, d2, d3, d4, d5, d6,
             sidx_all, isem, gsem, ssem):
    rows = [r0, r1, r2, r3, r4, r5, r6]
    didx = [d0, d1, d2, d3, d4, d5, d6]
    rbuf = rows[0]
    cid = lax.axis_index("c")
    sid = lax.axis_index("s")
    wid = cid * NS + sid
    base = wid * EPT

    def _zero_vec(i, _):
        r = i // (D // 16)
        c = i % (D // 16)
        rbuf[r, pl.ds(c * 16, 16)] = jnp.zeros((16,), jnp.float32)
        return _
    lax.fori_loop(0, RCHUNK * (D // 16), _zero_vec, None)

    def _zero_fire(j, _):
        k = sid + j * NS
        @pl.when(k < NROWCH)
        def _():
            pltpu.async_copy(rbuf, acc.at[pl.ds(k * RCHUNK, RCHUNK), :],
                             gsem.at[0])
        return _
    lax.fori_loop(0, (NROWCH + NS - 1) // NS, _zero_fire, None)

    def _zero_drain(j, _):
        k = sid + j * NS
        @pl.when(k < NROWCH)
        def _():
            pltpu.make_async_copy(
                rbuf, acc.at[pl.ds(k * RCHUNK, RCHUNK), :],
                gsem.at[0]).wait()
        return _
    lax.fori_loop(0, (NROWCH + NS - 1) // NS, _zero_drain, None)

    pltpu.sync_copy(src_hbm.at[pl.ds(base, EPT)], sidx_all)
    plsc.subcore_barrier()

    def _fire_didx(g, b):
        pltpu.async_copy(dst_hbm.at[pl.ds(base + g * C, C)], didx[b],
                         isem.at[b])

    def _wait_didx(b):
        pltpu.make_async_copy(
            dst_hbm.at[pl.ds(0, C)], didx[b], isem.at[b]).wait()

    def _fire_gather(g, b):
        pltpu.async_copy(
            hm_hbm.at[sidx_all.at[pl.ds(g * C, C)]], rows[b], gsem.at[b])

    def _wait_gather(g, b):
        pltpu.make_async_copy(
            hm_hbm.at[sidx_all.at[pl.ds(g * C, C)]], rows[b],
            gsem.at[b]).wait()

    def _fire_scatter(b):
        pltpu.async_copy(rows[b], acc.at[didx[b]], ssem.at[b], add=True)

    def _wait_scatter(b):
        pltpu.make_async_copy(
            rows[b], acc.at[didx[b]], ssem.at[b]).wait()

    for b in range(NBUF):
        _fire_didx(b, b)
        _fire_gather(b, b)

    def _outer(t, _):
        for b in range(NBUF):
            _wait_gather(t * NBUF + b, b)
            _wait_didx(b)
            _fire_scatter(b)
        for b in range(NBUF):
            g2 = (t + 1) * NBUF + b
            @pl.when(g2 < NCHUNK)
            def _():
                _wait_scatter(b)
                _fire_didx(g2, b)
                _fire_gather(g2, b)
        return _
    lax.fori_loop(0, NOUT, _outer, None)

    for b in range(NCHUNK - NOUT * NBUF):
        _wait_gather(NOUT * NBUF + b, b)
        _wait_didx(b)
        _fire_scatter(b)
    for b in range(NBUF):
        _wait_scatter(b)
    plsc.subcore_barrier()

    def _dump_fire(j, _):
        k = sid + j * NS
        @pl.when(k < NROWCH)
        def _():
            rr = k * RCHUNK
            pltpu.async_copy(acc.at[pl.ds(rr, RCHUNK), :],
                             out_hbm.at[cid, pl.ds(rr, RCHUNK), :],
                             gsem.at[0])
        return _
    lax.fori_loop(0, (NROWCH + NS - 1) // NS, _dump_fire, None)

    def _dump_drain(j, _):
        k = sid + j * NS
        @pl.when(k < NROWCH)
        def _():
            rr = k * RCHUNK
            pltpu.make_async_copy(
                acc.at[pl.ds(rr, RCHUNK), :],
                out_hbm.at[cid, pl.ds(rr, RCHUNK), :], gsem.at[0]).wait()
        return _
    lax.fori_loop(0, (NROWCH + NS - 1) // NS, _dump_drain, None)


def _sc_scatter(Hm, src_i32, dst_i32):
    mesh = plsc.VectorSubcoreMesh(core_axis_name="c", subcore_axis_name="s")
    f = functools.partial(
        pl.kernel,
        out_type=jax.ShapeDtypeStruct((NC, N, D), jnp.float32),
        mesh=mesh,
        scratch_types=[
            pltpu.VMEM_SHARED((N, D), jnp.float32),
        ] + [pltpu.VMEM((C, D), jnp.float32)] * NBUF + [
        ] + [pltpu.VMEM((C,), jnp.int32)] * NBUF + [
            pltpu.VMEM((EPT,), jnp.int32),
            pltpu.SemaphoreType.DMA((NBUF,)),
            pltpu.SemaphoreType.DMA((NBUF,)),
            pltpu.SemaphoreType.DMA((NBUF,)),
        ],
    )(_sc_body)
    return f(Hm, src_i32, dst_i32)


def _finish_body(h_ref, f_ref, p0_ref, p1_ref, w_ref, out_ref):
    agg = p0_ref[0] + p1_ref[0]
    y = lax.dot_general(agg, w_ref[...], (((1,), (1,)), ((), ())),
                        preferred_element_type=jnp.float32)
    out_ref[...] = h_ref[...] + f_ref[...] * y


def _finish(H, frag_col, P, W):
    grid = 5
    blk = N // grid
    return pl.pallas_call(
        _finish_body,
        grid=(grid,),
        in_specs=[
            pl.BlockSpec((blk, D), lambda i: (i, 0)),
            pl.BlockSpec((blk, 1), lambda i: (i, 0)),
            pl.BlockSpec((1, blk, D), lambda i: (0, i, 0)),
            pl.BlockSpec((1, blk, D), lambda i: (1, i, 0)),
            pl.BlockSpec((D, D), lambda i: (0, 0)),
        ],
        out_specs=pl.BlockSpec((blk, D), lambda i: (i, 0)),
        out_shape=jax.ShapeDtypeStruct((N, D), jnp.float32),
    )(H, frag_col, P, P, W)


def kernel(H, edge_index, frag_mask, W):
    frag_col = frag_mask.reshape(N, 1).astype(jnp.float32)
    ei = edge_index.astype(jnp.int32)
    Hm, src, dst = _masked_rows(H, frag_col, ei)
    P = _sc_scatter(Hm, src, dst)
    return _finish(H, frag_col, P, W)
